# baseline (speedup 1.0000x reference)
import jax
import jax.numpy as jnp
from jax.experimental import pallas as pl

_NUM_SLOTS = 256


def _body(slots_ref, x_ref, at_ref, bt_ref, o_ref):
    s = slots_ref[0]
    p = s.shape[-1]
    iot = jax.lax.broadcasted_iota(jnp.int32, (_NUM_SLOTS, p), 0)
    onehot = (iot == s).astype(jnp.bfloat16)
    a = jnp.dot(at_ref[...], onehot, preferred_element_type=jnp.float32)
    b = jnp.dot(bt_ref[...], onehot, preferred_element_type=jnp.float32)
    o_ref[0] = a * x_ref[0] + b


def kernel(x, slot_assign, alpha_table, beta_table):
    B, C, H, W = x.shape
    P = H * W
    S = alpha_table.shape[0]
    assert S == _NUM_SLOTS
    xr = x.reshape(B, C, P)
    slots = slot_assign.reshape(B, 1, P).astype(jnp.int32)
    at = alpha_table.T.astype(jnp.bfloat16)
    bt = beta_table.T.astype(jnp.bfloat16)

    CB = 128
    out = pl.pallas_call(
        _body,
        grid=(B, C // CB),
        in_specs=[
            pl.BlockSpec((1, 1, P), lambda b, c: (b, 0, 0)),
            pl.BlockSpec((1, CB, P), lambda b, c: (b, c, 0)),
            pl.BlockSpec((CB, S), lambda b, c: (c, 0)),
            pl.BlockSpec((CB, S), lambda b, c: (c, 0)),
        ],
        out_specs=pl.BlockSpec((1, CB, P), lambda b, c: (b, c, 0)),
        out_shape=jax.ShapeDtypeStruct((B, C, P), jnp.float32),
    )(slots, xr, at, bt)
    return out.reshape(B, C, H, W)
